# baseline (speedup 1.0000x reference)
import jax
import jax.numpy as jnp
import numpy as np
from jax.experimental import pallas as pl
from jax.experimental.pallas import tpu as pltpu

N = 10000
BM = 200
BK = 2048
NK = (N + BK - 1) // BK
NPAD = NK * BK

_IA, _JA, _SB = [], [], []
for _i in range(N // BM):
    _SB.append(len(_IA))
    for _j in range((BM * _i) // BK, NK):
        _IA.append(_i)
        _JA.append(_j)
TOT = len(_IA)


def _s0_kernel(x_ref, w0_ref, o_ref):
    o_ref[...] = jnp.dot(
        x_ref[...].astype(jnp.bfloat16), w0_ref[...].astype(jnp.bfloat16),
        preferred_element_type=jnp.float32).astype(jnp.bfloat16)


def _phase_a_kernel(sb_ref, adj_ref, s0_ref, b0_ref, w1_ref,
                    s1_ref, pout_ref, stash_ref,
                    rhs_ref, stg_ref, sem):
    i = pl.program_id(0)
    nhid = s0_ref.shape[1]
    jstart = (BM * i) // BK

    @pl.when(i == 0)
    def _init():
        rhs_ref[pl.ds(0, N), :nhid] = s0_ref[...]
        rhs_ref[pl.ds(N, NPAD - N), :nhid] = jnp.zeros(
            (NPAD - N, nhid), jnp.bfloat16)
        rhs_ref[:, nhid:] = jnp.zeros_like(rhs_ref[:, nhid:])

    col = jax.lax.broadcasted_iota(jnp.int32, (1, NPAD), 1)
    a = jnp.where(col < N, adj_ref[...], 0.0).astype(jnp.bfloat16)

    for jj in range(NK):
        @pl.when(jj >= jstart)
        def _stash(jj=jj):
            stg_ref[jj] = a[:, jj * BK:(jj + 1) * BK]
            slot = sb_ref[i] + jj - jstart
            pltpu.make_async_copy(
                stg_ref.at[jj], stash_ref.at[slot], sem).start()

    r = jnp.dot(a, rhs_ref[...], preferred_element_type=jnp.float32)
    pout_ref[...] = r[:, nhid:]
    h = jnp.maximum(r[:, :nhid] + b0_ref[...], 0.0).astype(jnp.bfloat16)
    s1_i = jnp.dot(h, w1_ref[...].astype(jnp.bfloat16),
                   preferred_element_type=jnp.float32).astype(jnp.bfloat16)
    s1_ref[...] = s1_i
    rhs_ref[pl.ds(i * BM, BM), nhid:] = s1_i

    for jj in range(NK):
        @pl.when(jj >= jstart)
        def _wait(jj=jj):
            slot = sb_ref[i] + jj - jstart
            pltpu.make_async_copy(
                stg_ref.at[jj], stash_ref.at[slot], sem).wait()


def _phase_b_kernel(ia_ref, ja_ref, stash_ref, s1_ref, pout_ref, b1_ref,
                    o_ref, acc_ref):
    s = pl.program_id(0)
    i = ia_ref[s]
    j = ja_ref[s]
    jstart = (BM * i) // BK

    @pl.when(j == jstart)
    def _init():
        acc_ref[...] = pout_ref[...]

    row = BK * j + jax.lax.broadcasted_iota(jnp.int32, (BK, 1), 0)
    s1m = jnp.where(row >= BM * i, s1_ref[...], jnp.bfloat16(0))

    acc_ref[...] += jnp.dot(stash_ref[0], s1m,
                            preferred_element_type=jnp.float32)

    @pl.when(j == NK - 1)
    def _fin():
        z = acc_ref[...] + b1_ref[...]
        m = jnp.max(z, axis=-1, keepdims=True)
        z = z - m
        lse = jnp.log(jnp.sum(jnp.exp(z), axis=-1, keepdims=True))
        o_ref[...] = z - lse


@jax.jit
def kernel(x, adj, W0, b0, W1, b1):
    nfeat = x.shape[1]
    nhid = W0.shape[1]
    ncls = W1.shape[1]

    s0 = pl.pallas_call(
        _s0_kernel,
        grid=(5,),
        in_specs=[
            pl.BlockSpec((N // 5, nfeat), lambda i: (i, 0)),
            pl.BlockSpec((nfeat, nhid), lambda i: (0, 0)),
        ],
        out_specs=pl.BlockSpec((N // 5, nhid), lambda i: (i, 0)),
        out_shape=jax.ShapeDtypeStruct((N, nhid), jnp.bfloat16),
    )(x, W0)

    sb = jnp.asarray(_SB, dtype=jnp.int32)
    ia = jnp.asarray(_IA, dtype=jnp.int32)
    ja = jnp.asarray(_JA, dtype=jnp.int32)

    s1, pout, stash = pl.pallas_call(
        _phase_a_kernel,
        grid_spec=pltpu.PrefetchScalarGridSpec(
            num_scalar_prefetch=1,
            grid=(N // BM,),
            in_specs=[
                pl.BlockSpec((BM, NPAD), lambda i, sbv: (i, 0)),
                pl.BlockSpec((N, nhid), lambda i, sbv: (0, 0)),
                pl.BlockSpec((1, nhid), lambda i, sbv: (0, 0)),
                pl.BlockSpec((nhid, ncls), lambda i, sbv: (0, 0)),
            ],
            out_specs=[
                pl.BlockSpec((BM, ncls), lambda i, sbv: (i, 0)),
                pl.BlockSpec((BM, ncls), lambda i, sbv: (i, 0)),
                pl.BlockSpec(memory_space=pltpu.MemorySpace.HBM),
            ],
            scratch_shapes=[
                pltpu.VMEM((NPAD, nhid + ncls), jnp.bfloat16),
                pltpu.VMEM((NK, BM, BK), jnp.bfloat16),
                pltpu.SemaphoreType.DMA,
            ],
        ),
        out_shape=[
            jax.ShapeDtypeStruct((N, ncls), jnp.bfloat16),
            jax.ShapeDtypeStruct((N, ncls), jnp.float32),
            jax.ShapeDtypeStruct((TOT, BM, BK), jnp.bfloat16),
        ],
        compiler_params=pltpu.CompilerParams(
            dimension_semantics=("arbitrary",)),
    )(sb, adj, s0, b0.reshape(1, nhid), W1)

    s1p = jnp.pad(s1, ((0, NPAD - N), (0, 0)))

    out = pl.pallas_call(
        _phase_b_kernel,
        grid_spec=pltpu.PrefetchScalarGridSpec(
            num_scalar_prefetch=2,
            grid=(TOT,),
            in_specs=[
                pl.BlockSpec((1, BM, BK), lambda s, iav, jav: (s, 0, 0)),
                pl.BlockSpec((BK, ncls), lambda s, iav, jav: (jav[s], 0)),
                pl.BlockSpec((BM, ncls), lambda s, iav, jav: (iav[s], 0)),
                pl.BlockSpec((1, ncls), lambda s, iav, jav: (0, 0)),
            ],
            out_specs=pl.BlockSpec((BM, ncls), lambda s, iav, jav: (iav[s], 0)),
            scratch_shapes=[pltpu.VMEM((BM, ncls), jnp.float32)],
        ),
        out_shape=jax.ShapeDtypeStruct((N, ncls), jnp.float32),
        compiler_params=pltpu.CompilerParams(
            dimension_semantics=("arbitrary",)),
    )(ia, ja, stash, s1p, pout, b1.reshape(1, ncls))

    return out


# speedup vs baseline: 1.4094x; 1.4094x over previous
import jax
import jax.numpy as jnp
from jax.experimental import pallas as pl
from jax.experimental.pallas import tpu as pltpu

N = 10000
BM = 400
GROUP = 5
GB = BM * GROUP
BK = 2048
NK = (N + BK - 1) // BK
NPAD = NK * BK

_IA, _JA = [], []
for _i in range(N // GB):
    for _j in range((GB * _i) // BK, NK):
        _IA.append(_i)
        _JA.append(_j)
TOT = len(_IA)


def _s0_kernel(x_ref, w0_ref, o_ref):
    o_ref[...] = jnp.dot(
        x_ref[...].astype(jnp.bfloat16), w0_ref[...].astype(jnp.bfloat16),
        preferred_element_type=jnp.float32).astype(jnp.bfloat16)


def _phase_a_kernel(adj_ref, s0_ref, b0_ref, w1_ref, s1_ref, pout_ref,
                    rhs_ref, s1v_ref):
    i = pl.program_id(0)
    nhid = s0_ref.shape[1]

    @pl.when(i == 0)
    def _init():
        rhs_ref[:, :nhid] = s0_ref[...]
        rhs_ref[:, nhid:] = jnp.zeros_like(rhs_ref[:, nhid:])

    @pl.when((i % GROUP == 0) & (i > 0))
    def _fold():
        base = (i // GROUP - 1) * GB
        rhs_ref[pl.ds(base, GB), nhid:] = s1v_ref[pl.ds(base, GB), :]

    a = adj_ref[...].astype(jnp.bfloat16)
    r = jnp.dot(a, rhs_ref[...], preferred_element_type=jnp.float32)
    pout_ref[...] = r[:, nhid:]
    h = jnp.maximum(r[:, :nhid] + b0_ref[...], 0.0).astype(jnp.bfloat16)
    s1_i = jnp.dot(h, w1_ref[...].astype(jnp.bfloat16),
                   preferred_element_type=jnp.float32).astype(jnp.bfloat16)
    s1_ref[...] = s1_i
    s1v_ref[pl.ds(i * BM, BM), :] = s1_i


def _phase_b_kernel(ia_ref, ja_ref, adj_ref, s1_ref, pout_ref, b1_ref,
                    o_ref, acc_ref):
    s = pl.program_id(0)
    i = ia_ref[s]
    j = ja_ref[s]
    jstart = (GB * i) // BK

    @pl.when(j == jstart)
    def _init():
        acc_ref[...] = pout_ref[...]

    row = BK * j + jax.lax.broadcasted_iota(jnp.int32, (BK, 1), 0)
    s1m = jnp.where(row >= GB * i, s1_ref[...], jnp.bfloat16(0))

    @pl.when(j < NK - 1)
    def _mid():
        acc_ref[...] += jnp.dot(adj_ref[...].astype(jnp.bfloat16), s1m,
                                preferred_element_type=jnp.float32)

    @pl.when(j == NK - 1)
    def _last():
        col = BK * j + jax.lax.broadcasted_iota(jnp.int32, (1, BK), 1)
        a = jnp.where(col < N, adj_ref[...], 0.0).astype(jnp.bfloat16)
        acc = acc_ref[...] + jnp.dot(a, s1m, preferred_element_type=jnp.float32)
        z = acc + b1_ref[...]
        m = jnp.max(z, axis=-1, keepdims=True)
        z = z - m
        lse = jnp.log(jnp.sum(jnp.exp(z), axis=-1, keepdims=True))
        o_ref[...] = z - lse


@jax.jit
def kernel(x, adj, W0, b0, W1, b1):
    nfeat = x.shape[1]
    nhid = W0.shape[1]
    ncls = W1.shape[1]

    s0 = pl.pallas_call(
        _s0_kernel,
        grid=(5,),
        in_specs=[
            pl.BlockSpec((N // 5, nfeat), lambda i: (i, 0)),
            pl.BlockSpec((nfeat, nhid), lambda i: (0, 0)),
        ],
        out_specs=pl.BlockSpec((N // 5, nhid), lambda i: (i, 0)),
        out_shape=jax.ShapeDtypeStruct((N, nhid), jnp.bfloat16),
    )(x, W0)

    s1, pout = pl.pallas_call(
        _phase_a_kernel,
        grid=(N // BM,),
        in_specs=[
            pl.BlockSpec((BM, N), lambda i: (i, 0)),
            pl.BlockSpec((N, nhid), lambda i: (0, 0)),
            pl.BlockSpec((1, nhid), lambda i: (0, 0)),
            pl.BlockSpec((nhid, ncls), lambda i: (0, 0)),
        ],
        out_specs=[
            pl.BlockSpec((BM, ncls), lambda i: (i, 0)),
            pl.BlockSpec((BM, ncls), lambda i: (i, 0)),
        ],
        out_shape=[
            jax.ShapeDtypeStruct((N, ncls), jnp.bfloat16),
            jax.ShapeDtypeStruct((N, ncls), jnp.float32),
        ],
        scratch_shapes=[
            pltpu.VMEM((N, nhid + ncls), jnp.bfloat16),
            pltpu.VMEM((N, ncls), jnp.bfloat16),
        ],
        compiler_params=pltpu.CompilerParams(
            dimension_semantics=("arbitrary",)),
    )(adj, s0, b0.reshape(1, nhid), W1)

    s1p = jnp.pad(s1, ((0, NPAD - N), (0, 0)))

    ia = jnp.asarray(_IA, dtype=jnp.int32)
    ja = jnp.asarray(_JA, dtype=jnp.int32)

    out = pl.pallas_call(
        _phase_b_kernel,
        grid_spec=pltpu.PrefetchScalarGridSpec(
            num_scalar_prefetch=2,
            grid=(TOT,),
            in_specs=[
                pl.BlockSpec((GB, BK), lambda s, iav, jav: (iav[s], jav[s])),
                pl.BlockSpec((BK, ncls), lambda s, iav, jav: (jav[s], 0)),
                pl.BlockSpec((GB, ncls), lambda s, iav, jav: (iav[s], 0)),
                pl.BlockSpec((1, ncls), lambda s, iav, jav: (0, 0)),
            ],
            out_specs=pl.BlockSpec((GB, ncls), lambda s, iav, jav: (iav[s], 0)),
            scratch_shapes=[pltpu.VMEM((GB, ncls), jnp.float32)],
        ),
        out_shape=jax.ShapeDtypeStruct((N, ncls), jnp.float32),
        compiler_params=pltpu.CompilerParams(
            dimension_semantics=("arbitrary",)),
    )(ia, ja, adj, s1p, pout, b1.reshape(1, ncls))

    return out
